# baseline (speedup 1.0000x reference)
import functools

import jax
import jax.numpy as jnp
from jax import lax
from jax.experimental import pallas as pl
from jax.experimental.pallas import tpu as pltpu
from jax.experimental.pallas import tpu_sc as plsc

N = 10000
E = 320000
D = 128

NUM_CORES = 2
NUM_SUBCORES = 16
NUM_WORKERS = NUM_CORES * NUM_SUBCORES
EDGES_PER_WORKER = E // NUM_WORKERS
CHUNK = 80
NUM_CHUNKS = EDGES_PER_WORKER // CHUNK
NBUF = 4
LBUF = 8
ROWS_PER_TILE = 624
COPY_ROWS = 16
LANES = 16


def _matmul(x, w):
    bm = 1000

    def body(x_ref, w_ref, o_ref):
        o_ref[...] = jnp.dot(x_ref[...], w_ref[...],
                             preferred_element_type=jnp.float32)

    return pl.pallas_call(
        body,
        grid=(N // bm,),
        in_specs=[
            pl.BlockSpec((bm, D), lambda i: (i, 0)),
            pl.BlockSpec((D, D), lambda i: (0, 0)),
        ],
        out_specs=pl.BlockSpec((bm, D), lambda i: (i, 0)),
        out_shape=jax.ShapeDtypeStruct((N, D), jnp.float32),
    )(x, w)


def _sum_partials(p):
    bm = 1000

    def body(p_ref, o_ref):
        o_ref[...] = p_ref[0] + p_ref[1]

    return pl.pallas_call(
        body,
        grid=(N // bm,),
        in_specs=[pl.BlockSpec((2, bm, D), lambda i: (0, i, 0))],
        out_specs=pl.BlockSpec((bm, D), lambda i: (i, 0)),
        out_shape=jax.ShapeDtypeStruct((N, D), jnp.float32),
    )(p)


def _make_sc_spmm():
    mesh = plsc.VectorSubcoreMesh(
        core_axis_name="c", subcore_axis_name="s",
        num_cores=NUM_CORES, num_subcores=NUM_SUBCORES)

    @functools.partial(
        pl.kernel,
        out_type=jax.ShapeDtypeStruct((NUM_CORES, N, D), jnp.float32),
        mesh=mesh,
        scratch_types=[
            [pltpu.VMEM((3, CHUNK), jnp.int32)] * LBUF,
            [pltpu.VMEM((CHUNK, D), jnp.float32)] * NBUF,
            pltpu.VMEM((COPY_ROWS, D), jnp.float32),
            pltpu.VMEM_SHARED((N, D), jnp.float32),
            [pltpu.SemaphoreType.DMA] * LBUF,
            [pltpu.SemaphoreType.DMA] * NBUF,
            [pltpu.SemaphoreType.DMA] * NBUF,
        ],
    )
    def spmm(z_hbm, pk_hbm, out_hbm,
             pkv, gbuf, zbuf, accum, lsem, gsem, ssem):
        c = lax.axis_index("c")
        s = lax.axis_index("s")

        zeros16 = jnp.zeros((LANES,), jnp.float32)
        for i in range(COPY_ROWS):
            for j in range(D // LANES):
                zbuf[i, pl.ds(j * LANES, LANES)] = zeros16

        row0 = s * ROWS_PER_TILE
        n_copy = (ROWS_PER_TILE + jnp.where(s == NUM_SUBCORES - 1, 16, 0)
                  ) // COPY_ROWS

        def zero_copy(t, _):
            pltpu.sync_copy(zbuf, accum.at[pl.ds(row0 + t * COPY_ROWS,
                                                 COPY_ROWS)])
            return 0

        lax.fori_loop(0, n_copy, zero_copy, 0)
        plsc.subcore_barrier()

        base_chunk = (c * NUM_SUBCORES + s) * NUM_CHUNKS

        def load_idx(p, sl):
            pltpu.async_copy(pk_hbm.at[base_chunk + p], pkv[sl], lsem[sl])

        def wait_idx(sl):
            pltpu.make_async_copy(pk_hbm.at[0], pkv[sl], lsem[sl]).wait()

        def start_gather(sl, lsl):
            pltpu.async_copy(z_hbm.at[pkv[lsl].at[0]], gbuf[sl], gsem[sl])

        def wait_gather(sl):
            pltpu.make_async_copy(z_hbm.at[pkv[0].at[0]], gbuf[sl],
                                  gsem[sl]).wait()

        def start_scatter(sl, lsl):
            pltpu.async_copy(gbuf[sl], accum.at[pkv[lsl].at[1]], ssem[sl],
                             add=True)

        def wait_scatter(sl):
            pltpu.make_async_copy(gbuf[sl], accum.at[pkv[0].at[1]],
                                  ssem[sl]).wait()

        def scale(sl, lsl):
            gb = gbuf[sl]
            pk = pkv[lsl]

            def grp(g, _):
                ew_vec = jax.lax.bitcast_convert_type(
                    pk[2, pl.ds(g * LANES, LANES)], jnp.float32)
                for e16 in range(LANES):
                    wv = jnp.take(ew_vec, jnp.full((LANES,), e16, jnp.int32))
                    e = g * LANES + e16
                    for j in range(D // LANES):
                        sl2 = pl.ds(j * LANES, LANES)
                        gb[e, sl2] = gb[e, sl2] * wv
                return 0

            lax.fori_loop(0, CHUNK // LANES, grp, 0)

        for p in range(4):
            load_idx(p, p)
        wait_idx(0)
        start_gather(0, 0)
        wait_idx(1)
        start_gather(1, 1)

        def chunk_step(a, b8, static_a=None):
            b = b8 % NBUF

            p_g = a + 2
            gsl = (b + 2) % NBUF
            lg = (b8 + 2) % LBUF
            if static_a is None or static_a + 2 < NUM_CHUNKS:
                if static_a is None and b8 < 2:
                    @pl.when(p_g >= NBUF)
                    def _():
                        wait_scatter(gsl)
                elif static_a is None or static_a + 2 >= NBUF:
                    wait_scatter(gsl)
                wait_idx(lg)
                start_gather(gsl, lg)

            p_l = a + 4
            lsl = (b8 + 4) % LBUF
            if static_a is None or static_a + 4 < NUM_CHUNKS:
                load_idx(p_l, lsl)

            wait_gather(b)
            scale(b, b8)
            start_scatter(b, b8)

        def steady(k8, _):
            for b8 in range(LBUF):
                chunk_step(k8 * LBUF + b8, b8)
            return 0

        n_steady = NUM_CHUNKS // LBUF
        lax.fori_loop(0, n_steady, steady, 0)

        for a in range(n_steady * LBUF, NUM_CHUNKS):
            chunk_step(a, a % LBUF, static_a=a)

        for b in range(NBUF):
            wait_scatter(b)
        plsc.subcore_barrier()

        def out_copy(t, _):
            r = row0 + t * COPY_ROWS
            pltpu.sync_copy(accum.at[pl.ds(r, COPY_ROWS)], zbuf)
            pltpu.sync_copy(zbuf, out_hbm.at[c, pl.ds(r, COPY_ROWS)])
            return 0

        lax.fori_loop(0, n_copy, out_copy, 0)

    return spmm


_sc_spmm = _make_sc_spmm()


@jax.jit
def kernel(x, edge_index, edge_weight, weight):
    z = _matmul(x, weight)
    ew_i32 = jax.lax.bitcast_convert_type(edge_weight, jnp.int32)
    pk = jnp.stack([edge_index[1], edge_index[0], ew_i32], axis=0)
    pk = pk.reshape(3, E // CHUNK, CHUNK).transpose(1, 0, 2)
    partials = _sc_spmm(z, pk)
    return _sum_partials(partials)


# speedup vs baseline: 1.1324x; 1.1324x over previous
import functools

import jax
import jax.numpy as jnp
from jax import lax
from jax.experimental import pallas as pl
from jax.experimental.pallas import tpu as pltpu
from jax.experimental.pallas import tpu_sc as plsc

N = 10000
E = 320000
D = 128

NUM_CORES = 2
NUM_SUBCORES = 16
NUM_WORKERS = NUM_CORES * NUM_SUBCORES
EDGES_PER_WORKER = E // NUM_WORKERS
CHUNK = 80
NUM_CHUNKS = EDGES_PER_WORKER // CHUNK
NBUF = 4
LBUF = 8
ROWS_PER_TILE = 624
COPY_ROWS = 16
LANES = 16


def _matmul(x, w):
    bm = 1000

    def body(x_ref, w_ref, o_ref):
        o_ref[...] = jnp.dot(x_ref[...], w_ref[...],
                             preferred_element_type=jnp.float32)

    return pl.pallas_call(
        body,
        grid=(N // bm,),
        in_specs=[
            pl.BlockSpec((bm, D), lambda i: (i, 0)),
            pl.BlockSpec((D, D), lambda i: (0, 0)),
        ],
        out_specs=pl.BlockSpec((bm, D), lambda i: (i, 0)),
        out_shape=jax.ShapeDtypeStruct((N, D), jnp.float32),
    )(x, w)


def _sum_partials(p):
    bm = 1000

    def body(p_ref, o_ref):
        o_ref[...] = p_ref[0] + p_ref[1]

    return pl.pallas_call(
        body,
        grid=(N // bm,),
        in_specs=[pl.BlockSpec((2, bm, D), lambda i: (0, i, 0))],
        out_specs=pl.BlockSpec((bm, D), lambda i: (i, 0)),
        out_shape=jax.ShapeDtypeStruct((N, D), jnp.float32),
    )(p)


def _make_sc_spmm():
    mesh = plsc.VectorSubcoreMesh(
        core_axis_name="c", subcore_axis_name="s",
        num_cores=NUM_CORES, num_subcores=NUM_SUBCORES)

    @functools.partial(
        pl.kernel,
        out_type=jax.ShapeDtypeStruct((NUM_CORES, N, D), jnp.float32),
        mesh=mesh,
        scratch_types=[
            [pltpu.VMEM((CHUNK,), jnp.int32)] * LBUF,
            [pltpu.VMEM((CHUNK,), jnp.int32)] * LBUF,
            [pltpu.VMEM((CHUNK,), jnp.float32)] * LBUF,
            [pltpu.VMEM((CHUNK, D), jnp.float32)] * NBUF,
            pltpu.VMEM((COPY_ROWS, D), jnp.float32),
            pltpu.VMEM_SHARED((N, D), jnp.float32),
            [pltpu.SemaphoreType.DMA] * LBUF,
            [pltpu.SemaphoreType.DMA] * NBUF,
            [pltpu.SemaphoreType.DMA] * NBUF,
            pltpu.SemaphoreType.DMA,
        ],
    )
    def spmm(z_hbm, cols_hbm, rows_hbm, ew_hbm, out_hbm,
             colv, rowv, ewv, gbuf, zbuf, accum, lsem, gsem, ssem, zsem):
        c = lax.axis_index("c")
        s = lax.axis_index("s")

        zeros16 = jnp.zeros((LANES,), jnp.float32)
        for i in range(COPY_ROWS):
            for j in range(D // LANES):
                zbuf[i, pl.ds(j * LANES, LANES)] = zeros16

        row0 = s * ROWS_PER_TILE
        n_copy = (ROWS_PER_TILE + jnp.where(s == NUM_SUBCORES - 1, 16, 0)
                  ) // COPY_ROWS

        def zero_copy(t, _):
            pltpu.async_copy(zbuf, accum.at[pl.ds(row0 + t * COPY_ROWS,
                                                  COPY_ROWS)], zsem)
            return 0

        def zero_wait(t, _):
            pltpu.make_async_copy(zbuf, accum.at[pl.ds(0, COPY_ROWS)],
                                  zsem).wait()
            return 0

        lax.fori_loop(0, n_copy, zero_copy, 0)
        lax.fori_loop(0, n_copy, zero_wait, 0)
        plsc.subcore_barrier()

        base = (c * NUM_SUBCORES + s) * EDGES_PER_WORKER

        def load_idx(p, sl):
            off = base + p * CHUNK
            pltpu.async_copy(cols_hbm.at[pl.ds(off, CHUNK)], colv[sl],
                             lsem[sl])
            pltpu.async_copy(rows_hbm.at[pl.ds(off, CHUNK)], rowv[sl],
                             lsem[sl])
            pltpu.async_copy(ew_hbm.at[pl.ds(off, CHUNK)], ewv[sl], lsem[sl])

        def wait_idx(sl):
            pltpu.make_async_copy(cols_hbm.at[pl.ds(0, CHUNK)], colv[sl],
                                  lsem[sl]).wait()
            pltpu.make_async_copy(rows_hbm.at[pl.ds(0, CHUNK)], rowv[sl],
                                  lsem[sl]).wait()
            pltpu.make_async_copy(ew_hbm.at[pl.ds(0, CHUNK)], ewv[sl],
                                  lsem[sl]).wait()

        def start_gather(sl, lsl):
            pltpu.async_copy(z_hbm.at[colv[lsl]], gbuf[sl], gsem[sl])

        def wait_gather(sl):
            pltpu.make_async_copy(z_hbm.at[colv[0]], gbuf[sl],
                                  gsem[sl]).wait()

        def start_scatter(sl, lsl):
            pltpu.async_copy(gbuf[sl], accum.at[rowv[lsl]], ssem[sl],
                             add=True)

        def wait_scatter(sl):
            pltpu.make_async_copy(gbuf[sl], accum.at[rowv[0]],
                                  ssem[sl]).wait()

        def scale(sl, lsl):
            gb = gbuf[sl]
            ew = ewv[lsl]

            def grp(g, _):
                ew_vec = ew[pl.ds(g * LANES, LANES)]
                for e16 in range(LANES):
                    wv = jnp.take(ew_vec, jnp.full((LANES,), e16, jnp.int32))
                    e = g * LANES + e16
                    for j in range(D // LANES):
                        sl2 = pl.ds(j * LANES, LANES)
                        gb[e, sl2] = gb[e, sl2] * wv
                return 0

            lax.fori_loop(0, CHUNK // LANES, grp, 0)

        for p in range(4):
            load_idx(p, p)
        wait_idx(0)
        start_gather(0, 0)
        wait_idx(1)
        start_gather(1, 1)

        def chunk_step(a, b8, static_a=None):
            b = b8 % NBUF

            p_g = a + 2
            gsl = (b + 2) % NBUF
            lg = (b8 + 2) % LBUF
            if static_a is None or static_a + 2 < NUM_CHUNKS:
                if static_a is None and b8 < 2:
                    @pl.when(p_g >= NBUF)
                    def _():
                        wait_scatter(gsl)
                elif static_a is None or static_a + 2 >= NBUF:
                    wait_scatter(gsl)
                wait_idx(lg)
                start_gather(gsl, lg)

            p_l = a + 4
            lsl = (b8 + 4) % LBUF
            if static_a is None or static_a + 4 < NUM_CHUNKS:
                load_idx(p_l, lsl)

            wait_gather(b)
            scale(b, b8)
            start_scatter(b, b8)

        def steady(k8, _):
            for b8 in range(LBUF):
                chunk_step(k8 * LBUF + b8, b8)
            return 0

        n_steady = NUM_CHUNKS // LBUF
        lax.fori_loop(0, n_steady, steady, 0)

        for a in range(n_steady * LBUF, NUM_CHUNKS):
            chunk_step(a, a % LBUF, static_a=a)

        for b in range(NBUF):
            wait_scatter(b)
        plsc.subcore_barrier()

        pltpu.async_copy(accum.at[pl.ds(row0, ROWS_PER_TILE)],
                         out_hbm.at[c, pl.ds(row0, ROWS_PER_TILE)], zsem)

        @pl.when(s == NUM_SUBCORES - 1)
        def _():
            tail = NUM_SUBCORES * ROWS_PER_TILE
            pltpu.async_copy(accum.at[pl.ds(tail, N - tail)],
                             out_hbm.at[c, pl.ds(tail, N - tail)], zsem)
            pltpu.make_async_copy(
                accum.at[pl.ds(0, N - tail)],
                out_hbm.at[c, pl.ds(0, N - tail)], zsem).wait()

        pltpu.make_async_copy(
            accum.at[pl.ds(0, ROWS_PER_TILE)],
            out_hbm.at[c, pl.ds(0, ROWS_PER_TILE)], zsem).wait()

    return spmm


_sc_spmm = _make_sc_spmm()


@jax.jit
def kernel(x, edge_index, edge_weight, weight):
    z = _matmul(x, weight)
    rows = edge_index[0]
    cols = edge_index[1]
    partials = _sc_spmm(z, cols, rows, edge_weight)
    return _sum_partials(partials)


# speedup vs baseline: 1.1907x; 1.0516x over previous
import functools

import jax
import jax.numpy as jnp
from jax import lax
from jax.experimental import pallas as pl
from jax.experimental.pallas import tpu as pltpu
from jax.experimental.pallas import tpu_sc as plsc

N = 10000
E = 320000
D = 128

NUM_CORES = 2
NUM_SUBCORES = 16
NUM_WORKERS = NUM_CORES * NUM_SUBCORES
EDGES_PER_WORKER = E // NUM_WORKERS
CHUNK = 80
NUM_CHUNKS = EDGES_PER_WORKER // CHUNK
NUM_CHUNKS_TOT = E // CHUNK
NBUF = 4
IBLK = 5
ROWS_PER_TILE = 624
COPY_ROWS = 16
LANES = 16


def _matmul(x, w):
    bm = 1000

    def body(x_ref, w_ref, o_ref):
        o_ref[...] = jnp.dot(x_ref[...], w_ref[...],
                             preferred_element_type=jnp.float32)

    return pl.pallas_call(
        body,
        grid=(N // bm,),
        in_specs=[
            pl.BlockSpec((bm, D), lambda i: (i, 0)),
            pl.BlockSpec((D, D), lambda i: (0, 0)),
        ],
        out_specs=pl.BlockSpec((bm, D), lambda i: (i, 0)),
        out_shape=jax.ShapeDtypeStruct((N, D), jnp.float32),
    )(x, w)


def _sum_partials(p):
    bm = 1000

    def body(p_ref, o_ref):
        o_ref[...] = p_ref[0] + p_ref[1]

    return pl.pallas_call(
        body,
        grid=(N // bm,),
        in_specs=[pl.BlockSpec((2, bm, D), lambda i: (0, i, 0))],
        out_specs=pl.BlockSpec((bm, D), lambda i: (i, 0)),
        out_shape=jax.ShapeDtypeStruct((N, D), jnp.float32),
    )(p)


def _make_sc_spmm():
    mesh = plsc.VectorSubcoreMesh(
        core_axis_name="c", subcore_axis_name="s",
        num_cores=NUM_CORES, num_subcores=NUM_SUBCORES)

    @functools.partial(
        pl.kernel,
        out_type=jax.ShapeDtypeStruct((NUM_CORES, N, D), jnp.float32),
        mesh=mesh,
        scratch_types=[
            pltpu.VMEM((2 * IBLK, 1, CHUNK), jnp.int32),
            pltpu.VMEM((2 * IBLK, 1, CHUNK), jnp.int32),
            pltpu.VMEM((2 * IBLK, 1, CHUNK), jnp.float32),
            [pltpu.VMEM((CHUNK, D), jnp.float32)] * NBUF,
            pltpu.VMEM((COPY_ROWS, D), jnp.float32),
            pltpu.VMEM_SHARED((N, D), jnp.float32),
            pltpu.SemaphoreType.DMA,
            [pltpu.SemaphoreType.DMA] * NBUF,
            [pltpu.SemaphoreType.DMA] * NBUF,
            pltpu.SemaphoreType.DMA,
        ],
    )
    def spmm(z_hbm, ei3, ew3, out_hbm,
             cb, rb, eb, gbuf, zbuf, accum, bsem, gsem, ssem, zsem):
        c = lax.axis_index("c")
        s = lax.axis_index("s")

        zeros16 = jnp.zeros((LANES,), jnp.float32)
        for i in range(COPY_ROWS):
            for j in range(D // LANES):
                zbuf[i, pl.ds(j * LANES, LANES)] = zeros16

        row0 = s * ROWS_PER_TILE
        n_copy = (ROWS_PER_TILE + jnp.where(s == NUM_SUBCORES - 1, 16, 0)
                  ) // COPY_ROWS

        def zero_copy(t, _):
            pltpu.async_copy(zbuf, accum.at[pl.ds(row0 + t * COPY_ROWS,
                                                  COPY_ROWS)], zsem)
            return 0

        def zero_wait(t, _):
            pltpu.make_async_copy(zbuf, accum.at[pl.ds(0, COPY_ROWS)],
                                  zsem).wait()
            return 0

        lax.fori_loop(0, n_copy, zero_copy, 0)

        base_chunk = (c * NUM_SUBCORES + s) * NUM_CHUNKS

        def fire_block(blk):
            half = lax.rem(blk, 2) * IBLK
            src = base_chunk + blk * IBLK
            pltpu.async_copy(ei3.at[pl.ds(NUM_CHUNKS_TOT + src, IBLK)],
                             cb.at[pl.ds(half, IBLK)], bsem)
            pltpu.async_copy(ei3.at[pl.ds(src, IBLK)],
                             rb.at[pl.ds(half, IBLK)], bsem)
            pltpu.async_copy(ew3.at[pl.ds(src, IBLK)],
                             eb.at[pl.ds(half, IBLK)], bsem)

        def wait_block():
            for _ in range(2):
                pltpu.make_async_copy(ei3.at[pl.ds(0, IBLK)],
                                      cb.at[pl.ds(0, IBLK)], bsem).wait()
            pltpu.make_async_copy(ew3.at[pl.ds(0, IBLK)],
                                  eb.at[pl.ds(0, IBLK)], bsem).wait()

        def start_gather(sl, a):
            pltpu.async_copy(z_hbm.at[cb.at[lax.rem(a, 2 * IBLK), 0]],
                             gbuf[sl], gsem[sl])

        def wait_gather(sl):
            pltpu.make_async_copy(z_hbm.at[cb.at[0, 0]], gbuf[sl],
                                  gsem[sl]).wait()

        def start_scatter(sl, a):
            pltpu.async_copy(gbuf[sl],
                             accum.at[rb.at[lax.rem(a, 2 * IBLK), 0]],
                             ssem[sl], add=True)

        def wait_scatter(sl):
            pltpu.make_async_copy(gbuf[sl], accum.at[rb.at[0, 0]],
                                  ssem[sl]).wait()

        def scale(sl, a):
            gb = gbuf[sl]
            r10 = lax.rem(a, 2 * IBLK)

            def grp(g, _):
                ew_vec = eb[r10, 0, pl.ds(g * LANES, LANES)]
                for e16 in range(LANES):
                    wv = jnp.take(ew_vec, jnp.full((LANES,), e16, jnp.int32))
                    e = g * LANES + e16
                    for j in range(D // LANES):
                        sl2 = pl.ds(j * LANES, LANES)
                        gb[e, sl2] = gb[e, sl2] * wv
                return 0

            lax.fori_loop(0, CHUNK // LANES, grp, 0)

        fire_block(0)
        wait_block()
        lax.fori_loop(0, n_copy, zero_wait, 0)
        plsc.subcore_barrier()
        start_gather(0, 0)
        start_gather(1, 1)

        def chunk_step(a, b, static_a=None):
            if static_a is None:
                @pl.when((lax.rem(a, IBLK) == 3) & (a < NUM_CHUNKS - 4))
                def _():
                    wait_block()

            p_g = a + 2
            gsl = (b + 2) % NBUF
            if static_a is None or static_a + 2 < NUM_CHUNKS:
                if static_a is None and b < 2:
                    @pl.when(p_g >= NBUF)
                    def _():
                        wait_scatter(gsl)
                else:
                    wait_scatter(gsl)
                start_gather(gsl, p_g)

            if static_a is None:
                @pl.when((lax.rem(a, IBLK) == 1) & (a < NUM_CHUNKS - 4))
                def _():
                    fire_block(a // IBLK + 1)

            wait_gather(b)
            scale(b, a)
            start_scatter(b, a)

        def steady(k4, _):
            for b in range(NBUF):
                chunk_step(k4 * NBUF + b, b)
            return 0

        n_steady = (NUM_CHUNKS - NBUF - 1) // NBUF
        lax.fori_loop(0, n_steady, steady, 0)

        for a in range(n_steady * NBUF, NUM_CHUNKS):
            chunk_step(a, a % NBUF, static_a=a)

        for b in range(NBUF):
            wait_scatter(b)
        plsc.subcore_barrier()

        pltpu.async_copy(accum.at[pl.ds(row0, ROWS_PER_TILE)],
                         out_hbm.at[c, pl.ds(row0, ROWS_PER_TILE)], zsem)

        @pl.when(s == NUM_SUBCORES - 1)
        def _():
            tail = NUM_SUBCORES * ROWS_PER_TILE
            pltpu.async_copy(accum.at[pl.ds(tail, N - tail)],
                             out_hbm.at[c, pl.ds(tail, N - tail)], zsem)
            pltpu.make_async_copy(
                accum.at[pl.ds(0, N - tail)],
                out_hbm.at[c, pl.ds(0, N - tail)], zsem).wait()

        pltpu.make_async_copy(
            accum.at[pl.ds(0, ROWS_PER_TILE)],
            out_hbm.at[c, pl.ds(0, ROWS_PER_TILE)], zsem).wait()

    return spmm


_sc_spmm = _make_sc_spmm()


@jax.jit
def kernel(x, edge_index, edge_weight, weight):
    z = _matmul(x, weight)
    ei3 = edge_index.reshape(2 * NUM_CHUNKS_TOT, 1, CHUNK)
    ew3 = edge_weight.reshape(NUM_CHUNKS_TOT, 1, CHUNK)
    partials = _sc_spmm(z, ei3, ew3)
    return _sum_partials(partials)
